# baseline (speedup 1.0000x reference)
import functools

import jax
import jax.numpy as jnp
from jax import lax
from jax.experimental import pallas as pl
from jax.experimental.pallas import tpu as pltpu
from jax.experimental.pallas import tpu_sc as plsc

D_MODEL = 2048
D_SAE = 16384
N_TOK = 4096
TOPK = 64


def _encode_body(x_ref, w_ref, b_ref, out_ref):
    acc = jax.lax.dot_general(
        x_ref[...], w_ref[...],
        dimension_numbers=(((1,), (1,)), ((), ())),
        preferred_element_type=jnp.float32,
    )
    out_ref[...] = acc + b_ref[...]


def _encode(x, W_enc, b_enc, tb=512, sb=2048):
    grid = (D_SAE // sb, N_TOK // tb)
    return pl.pallas_call(
        _encode_body,
        grid=grid,
        in_specs=[
            pl.BlockSpec((tb, D_MODEL), lambda j, i: (i, 0)),
            pl.BlockSpec((sb, D_MODEL), lambda j, i: (j, 0)),
            pl.BlockSpec((1, sb), lambda j, i: (0, j)),
        ],
        out_specs=pl.BlockSpec((tb, sb), lambda j, i: (i, j)),
        out_shape=jax.ShapeDtypeStruct((N_TOK, D_SAE), jnp.float32),
        compiler_params=pltpu.CompilerParams(
            dimension_semantics=("arbitrary", "arbitrary"),
        ),
    )(x, W_enc, b_enc.reshape(1, D_SAE))


def _mono_u32(v):
    b = pltpu.bitcast(v, jnp.uint32)
    return jnp.where(b < jnp.uint32(0x80000000),
                     b ^ jnp.uint32(0x80000000),
                     ~b)


def _unmono_f32(u):
    b = jnp.where(u >= jnp.uint32(0x80000000), u ^ jnp.uint32(0x80000000), ~u)
    return pltpu.bitcast(b, jnp.float32)


_NW = 32
_RPW = N_TOK // _NW
_NVEC = D_SAE // 16
_CAP = 2048


def _mono_vec(v):
    b = plsc.bitcast(v, jnp.uint32)
    return jnp.where(b < jnp.uint32(0x80000000), b ^ jnp.uint32(0x80000000), ~b)


def _smax(x):
    return lax.reduce_max(x, axes=(0,))


def _popcnt(mask):
    return _smax(plsc.all_reduce_population_count(mask))


def _sc_thresh_kernel(lat_hbm, out_hbm, rowbuf, hist, cand, thrbuf):
    wid = lax.axis_index("s") * 2 + lax.axis_index("c")
    base = wid * _RPW
    iota = lax.broadcasted_iota(jnp.int32, (16,), 0)
    zeros16 = jnp.zeros((16,), jnp.int32)

    def zero_hist(i, _):
        hist[pl.ds(i * 16, 16)] = zeros16
        return 0

    lax.fori_loop(0, 65536 // 16, zero_hist, 0)

    def do_row(row, thr_acc):
        pltpu.sync_copy(lat_hbm.at[base + row], rowbuf)

        def p1(i, mx):
            u = _mono_vec(rowbuf[pl.ds(i * 16, 16)])
            bucket = (u >> jnp.uint32(16)).astype(jnp.int32)
            plsc.addupdate_scatter(hist, [bucket], jnp.ones((16,), jnp.int32))
            return jnp.maximum(mx, bucket)

        mxb = lax.fori_loop(0, _NVEC, p1, zeros16)
        start_vec = _smax(mxb) >> 4

        def scan_cond(st):
            return st[4] == 0

        def scan_body(st):
            c, vi, t16, cgt, _found = st
            h = hist[pl.ds(vi * 16, 16)]
            p = plsc.cumsum(h)
            s = _smax(p)
            cond_vec = (c + s - p + h) >= TOPK
            r = _popcnt(cond_vec)
            found = (r > 0).astype(jnp.int32)
            pm1 = _smax(jnp.where(iota == r - 1, p, 0))
            t16_new = jnp.where(r > 0, vi * 16 + r - 1, t16)
            cgt_new = jnp.where(r > 0, c + s - pm1, cgt)
            return (c + s, vi - 1, t16_new, cgt_new, found)

        _, _, t16, cgt, _ = lax.while_loop(
            scan_cond, scan_body,
            (jnp.int32(0), start_vec, jnp.int32(0), jnp.int32(0), jnp.int32(0)))
        rank = TOPK - cgt

        def p2(i, off):
            u = _mono_vec(rowbuf[pl.ds(i * 16, 16)])
            bucket = (u >> jnp.uint32(16)).astype(jnp.int32)
            plsc.store_scatter(hist, [bucket], zeros16)
            eq = bucket == t16
            cnt = _popcnt(eq)

            def collect(o):
                eqi = eq.astype(jnp.int32)
                pos = jnp.minimum(o + plsc.cumsum(eqi) - eqi, _CAP - 1)
                low = (u & jnp.uint32(0xFFFF)).astype(jnp.int32)
                plsc.store_scatter(cand, [pos], low, mask=eq)
                return o + cnt

            return lax.cond(cnt > 0, collect, lambda o: o, off)

        off = lax.fori_loop(0, _NVEC, p2, jnp.int32(0))

        nvec = (off + 15) >> 4

        def sbit(b, lo):
            mid = lo | (jnp.int32(1) << (15 - b))

            def count_vec(j, c):
                vec = cand[pl.ds(j * 16, 16)]
                m = ((j * 16 + iota) < off) & (vec >= mid)
                return c + _popcnt(m)

            cnt = lax.fori_loop(0, nvec, count_vec, jnp.int32(0))
            return jnp.where(cnt >= rank, mid, lo)

        low = lax.fori_loop(0, 16, sbit, jnp.int32(0))

        u_thr = (jnp.broadcast_to(t16.astype(jnp.uint32), (16,)) << jnp.uint32(16)) \
            | jnp.broadcast_to(low.astype(jnp.uint32), (16,))
        bits = jnp.where(u_thr >= jnp.uint32(0x80000000),
                         u_thr ^ jnp.uint32(0x80000000), ~u_thr)
        thr_f = plsc.bitcast(bits, jnp.float32)
        thr_acc = jnp.where(iota == (row & 15), thr_f, thr_acc)

        @pl.when((row & 15) == 15)
        def _flush():
            thrbuf[pl.ds((row >> 4) * 16, 16)] = thr_acc

        return thr_acc

    lax.fori_loop(0, _RPW, do_row, jnp.zeros((16,), jnp.float32))
    pltpu.sync_copy(thrbuf, out_hbm.at[pl.ds(base, _RPW)])


def _sc_thresholds(latents):
    mesh = plsc.VectorSubcoreMesh(core_axis_name="c", subcore_axis_name="s")
    f = pl.kernel(
        _sc_thresh_kernel,
        out_type=jax.ShapeDtypeStruct((N_TOK,), jnp.float32),
        mesh=mesh,
        scratch_types=[
            pltpu.VMEM((D_SAE,), jnp.float32),
            pltpu.VMEM((65536,), jnp.int32),
            pltpu.VMEM((_CAP,), jnp.int32),
            pltpu.VMEM((_RPW,), jnp.float32),
        ],
        compiler_params=pltpu.CompilerParams(needs_layout_passes=False),
    )
    return f(latents).reshape(N_TOK, 1)


def _thresh_body(lat_ref, thr_ref):
    mono = _mono_u32(lat_ref[...])
    tb = mono.shape[0]
    lo0 = jnp.zeros((tb, 1), dtype=jnp.uint32)

    def step(i, lo):
        bit = jnp.uint32(1) << (jnp.uint32(31) - jnp.uint32(i))
        mid = lo | bit
        cnt = jnp.sum((mono >= mid).astype(jnp.int32), axis=1, keepdims=True)
        return jnp.where(cnt >= TOPK, mid, lo)

    lo = jax.lax.fori_loop(0, 32, step, lo0)
    thr_ref[...] = _unmono_f32(lo)


def _thresholds(latents, tb=128):
    return pl.pallas_call(
        _thresh_body,
        grid=(N_TOK // tb,),
        in_specs=[pl.BlockSpec((tb, D_SAE), lambda i: (i, 0))],
        out_specs=pl.BlockSpec((tb, 1), lambda i: (i, 0)),
        out_shape=jax.ShapeDtypeStruct((N_TOK, 1), jnp.float32),
    )(latents)


def _decode_body(lat_ref, thr_ref, w_ref, b_ref, sparse_ref, recon_ref):
    k = pl.program_id(1)
    sparse = jnp.where(lat_ref[...] >= thr_ref[...], lat_ref[...], 0.0)
    sparse_ref[...] = sparse
    partial = jax.lax.dot_general(
        sparse.astype(jnp.bfloat16), w_ref[...],
        dimension_numbers=(((1,), (1,)), ((), ())),
        preferred_element_type=jnp.float32,
    )

    @pl.when(k == 0)
    def _init():
        recon_ref[...] = partial + b_ref[...]

    @pl.when(k != 0)
    def _acc():
        recon_ref[...] += partial


def _decode(latents, thr, W_dec, b_dec, tb=512, kb=2048):
    grid = (N_TOK // tb, D_SAE // kb)
    return pl.pallas_call(
        _decode_body,
        grid=grid,
        in_specs=[
            pl.BlockSpec((tb, kb), lambda i, k: (i, k)),
            pl.BlockSpec((tb, 1), lambda i, k: (i, 0)),
            pl.BlockSpec((D_MODEL, kb), lambda i, k: (0, k)),
            pl.BlockSpec((1, D_MODEL), lambda i, k: (0, 0)),
        ],
        out_specs=[
            pl.BlockSpec((tb, kb), lambda i, k: (i, k)),
            pl.BlockSpec((tb, D_MODEL), lambda i, k: (i, 0)),
        ],
        out_shape=[
            jax.ShapeDtypeStruct((N_TOK, D_SAE), jnp.float32),
            jax.ShapeDtypeStruct((N_TOK, D_MODEL), jnp.float32),
        ],
        compiler_params=pltpu.CompilerParams(
            dimension_semantics=("arbitrary", "arbitrary"),
        ),
    )(latents, thr, W_dec.astype(jnp.bfloat16), b_dec.reshape(1, D_MODEL))


@jax.jit
def kernel(x, W_enc, b_enc, W_dec, b_dec):
    latents = _encode(x, W_enc, b_enc)
    thr = _sc_thresholds(latents)
    sparse_latents, recon = _decode(latents, thr, W_dec, b_dec)
    return recon, sparse_latents


# speedup vs baseline: 4.3752x; 4.3752x over previous
import functools

import jax
import jax.numpy as jnp
from jax import lax
from jax.experimental import pallas as pl
from jax.experimental.pallas import tpu as pltpu
from jax.experimental.pallas import tpu_sc as plsc

D_MODEL = 2048
D_SAE = 16384
N_TOK = 4096
TOPK = 64


def _encode_body(x_ref, w_ref, b_ref, out_ref):
    acc = jax.lax.dot_general(
        x_ref[...], w_ref[...],
        dimension_numbers=(((1,), (1,)), ((), ())),
        preferred_element_type=jnp.float32,
    )
    out_ref[...] = acc + b_ref[...]


def _encode(x, W_enc, b_enc, tb=512, sb=2048):
    grid = (D_SAE // sb, N_TOK // tb)
    return pl.pallas_call(
        _encode_body,
        grid=grid,
        in_specs=[
            pl.BlockSpec((tb, D_MODEL), lambda j, i: (i, 0)),
            pl.BlockSpec((sb, D_MODEL), lambda j, i: (j, 0)),
            pl.BlockSpec((1, sb), lambda j, i: (0, j)),
        ],
        out_specs=pl.BlockSpec((tb, sb), lambda j, i: (i, j)),
        out_shape=jax.ShapeDtypeStruct((N_TOK, D_SAE), jnp.float32),
        compiler_params=pltpu.CompilerParams(
            dimension_semantics=("arbitrary", "arbitrary"),
        ),
    )(x, W_enc, b_enc.reshape(1, D_SAE))


def _mono_u32(v):
    b = pltpu.bitcast(v, jnp.uint32)
    return jnp.where(b < jnp.uint32(0x80000000),
                     b ^ jnp.uint32(0x80000000),
                     ~b)


def _unmono_f32(u):
    b = jnp.where(u >= jnp.uint32(0x80000000), u ^ jnp.uint32(0x80000000), ~u)
    return pltpu.bitcast(b, jnp.float32)


_NW = 32
_RPW = N_TOK // _NW
_NVEC = D_SAE // 16
_CAP = 2048


def _mono_vec(v):
    b = plsc.bitcast(v, jnp.uint32)
    return jnp.where(b < jnp.uint32(0x80000000), b ^ jnp.uint32(0x80000000), ~b)


def _smax(x):
    return lax.reduce_max(x, axes=(0,))


def _popcnt(mask):
    return _smax(plsc.all_reduce_population_count(mask))


def _scan_hist(hist_ref, start_vec, kneed, iota):
    def scan_cond(st):
        return st[4] == 0

    def scan_body(st):
        c, vi, bsel, cgt, _found = st
        h = hist_ref[pl.ds(vi * 16, 16)]
        p = plsc.cumsum(h)
        s = _smax(p)
        cond_vec = (c + s - p + h) >= kneed
        r = _popcnt(cond_vec)
        pm1 = _smax(jnp.where(iota == r - 1, p, 0))
        bsel_new = jnp.where(r > 0, vi * 16 + r - 1, bsel)
        cgt_new = jnp.where(r > 0, c + s - pm1, cgt)
        return (c + s, vi - 1, bsel_new, cgt_new, (r > 0).astype(jnp.int32))

    _, _, bsel, cgt, _ = lax.while_loop(
        scan_cond, scan_body,
        (jnp.int32(0), start_vec, jnp.int32(0), jnp.int32(0), jnp.int32(0)))
    return bsel, cgt


def _sc_thresh_kernel(lat_hbm, out_hbm, buf0, buf1, hist, hist2, hist3,
                      thrbuf, sem0, sem1):
    wid = lax.axis_index("s") * 2 + lax.axis_index("c")
    base = wid * _RPW
    iota = lax.broadcasted_iota(jnp.int32, (16,), 0)
    zeros16 = jnp.zeros((16,), jnp.int32)
    ones16 = jnp.ones((16,), jnp.int32)

    @plsc.parallel_loop(0, 65536, step=16, unroll=8)
    def _zh(i):
        hist[pl.ds(i, 16)] = zeros16

    for h in (hist2, hist3):
        for i in range(16):
            h[pl.ds(i * 16, 16)] = zeros16

    def process(buf, row, thr_acc):
        @plsc.parallel_loop(0, D_SAE, step=16, unroll=8, carry=zeros16)
        def mxb(i, mx):
            u = _mono_vec(buf[pl.ds(i, 16)])
            bucket = (u >> jnp.uint32(16)).astype(jnp.int32)
            plsc.addupdate_scatter(hist, [bucket], ones16)
            return jnp.maximum(mx, bucket)

        t16, cgt = _scan_hist(hist, _smax(mxb) >> 4, TOPK, iota)
        rank = TOPK - cgt

        @plsc.parallel_loop(0, D_SAE, step=16, unroll=8)
        def _p2(i):
            u = _mono_vec(buf[pl.ds(i, 16)])
            bucket = (u >> jnp.uint32(16)).astype(jnp.int32)
            plsc.store_scatter(hist, [bucket], zeros16)
            b2 = ((u >> jnp.uint32(8)) & jnp.uint32(0xFF)).astype(jnp.int32)
            plsc.addupdate_scatter(hist2, [b2], ones16, mask=bucket == t16)

        b8, cgt2 = _scan_hist(hist2, jnp.int32(15), rank, iota)
        rank3 = rank - cgt2

        @plsc.parallel_loop(0, D_SAE, step=16, unroll=8)
        def _p3(i):
            u = _mono_vec(buf[pl.ds(i, 16)])
            bucket = (u >> jnp.uint32(16)).astype(jnp.int32)
            b2 = ((u >> jnp.uint32(8)) & jnp.uint32(0xFF)).astype(jnp.int32)
            low8 = (u & jnp.uint32(0xFF)).astype(jnp.int32)
            plsc.addupdate_scatter(hist3, [low8], ones16,
                                   mask=(bucket == t16) & (b2 == b8))

        b0, _ = _scan_hist(hist3, jnp.int32(15), rank3, iota)

        for h in (hist2, hist3):
            for i in range(16):
                h[pl.ds(i * 16, 16)] = zeros16

        u_thr = (jnp.broadcast_to(t16.astype(jnp.uint32), (16,)) << jnp.uint32(16)) \
            | (jnp.broadcast_to(b8.astype(jnp.uint32), (16,)) << jnp.uint32(8)) \
            | jnp.broadcast_to(b0.astype(jnp.uint32), (16,))
        bits = jnp.where(u_thr >= jnp.uint32(0x80000000),
                         u_thr ^ jnp.uint32(0x80000000), ~u_thr)
        thr_f = plsc.bitcast(bits, jnp.float32)
        thr_acc = jnp.where(iota == (row & 15), thr_f, thr_acc)

        @pl.when((row & 15) == 15)
        def _flush():
            thrbuf[pl.ds((row >> 4) * 16, 16)] = thr_acc

        return thr_acc

    pltpu.async_copy(lat_hbm.at[base], buf0, sem0)

    def do_pair(i, thr_acc):
        r0 = base + 2 * i
        pltpu.async_copy(lat_hbm.at[r0 + 1], buf1, sem1)
        pltpu.make_async_copy(lat_hbm.at[r0], buf0, sem0).wait()
        thr_acc = process(buf0, 2 * i, thr_acc)

        @pl.when(i < _RPW // 2 - 1)
        def _prefetch():
            pltpu.async_copy(lat_hbm.at[r0 + 2], buf0, sem0)

        pltpu.make_async_copy(lat_hbm.at[r0 + 1], buf1, sem1).wait()
        return process(buf1, 2 * i + 1, thr_acc)

    lax.fori_loop(0, _RPW // 2, do_pair, jnp.zeros((16,), jnp.float32))
    pltpu.sync_copy(thrbuf, out_hbm.at[pl.ds(base, _RPW)])


def _sc_thresholds(latents):
    mesh = plsc.VectorSubcoreMesh(core_axis_name="c", subcore_axis_name="s")
    f = pl.kernel(
        _sc_thresh_kernel,
        out_type=jax.ShapeDtypeStruct((N_TOK,), jnp.float32),
        mesh=mesh,
        scratch_types=[
            pltpu.VMEM((D_SAE,), jnp.float32),
            pltpu.VMEM((D_SAE,), jnp.float32),
            pltpu.VMEM((65536,), jnp.int32),
            pltpu.VMEM((256,), jnp.int32),
            pltpu.VMEM((256,), jnp.int32),
            pltpu.VMEM((_RPW,), jnp.float32),
            pltpu.SemaphoreType.DMA,
            pltpu.SemaphoreType.DMA,
        ],
        compiler_params=pltpu.CompilerParams(needs_layout_passes=False),
    )
    return f(latents).reshape(N_TOK, 1)


def _thresh_body(lat_ref, thr_ref):
    mono = _mono_u32(lat_ref[...])
    tb = mono.shape[0]
    lo0 = jnp.zeros((tb, 1), dtype=jnp.uint32)

    def step(i, lo):
        bit = jnp.uint32(1) << (jnp.uint32(31) - jnp.uint32(i))
        mid = lo | bit
        cnt = jnp.sum((mono >= mid).astype(jnp.int32), axis=1, keepdims=True)
        return jnp.where(cnt >= TOPK, mid, lo)

    lo = jax.lax.fori_loop(0, 32, step, lo0)
    thr_ref[...] = _unmono_f32(lo)


def _thresholds(latents, tb=128):
    return pl.pallas_call(
        _thresh_body,
        grid=(N_TOK // tb,),
        in_specs=[pl.BlockSpec((tb, D_SAE), lambda i: (i, 0))],
        out_specs=pl.BlockSpec((tb, 1), lambda i: (i, 0)),
        out_shape=jax.ShapeDtypeStruct((N_TOK, 1), jnp.float32),
    )(latents)


def _decode_body(lat_ref, thr_ref, w_ref, b_ref, sparse_ref, recon_ref):
    k = pl.program_id(1)
    sparse = jnp.where(lat_ref[...] >= thr_ref[...], lat_ref[...], 0.0)
    sparse_ref[...] = sparse
    partial = jax.lax.dot_general(
        sparse.astype(jnp.bfloat16), w_ref[...],
        dimension_numbers=(((1,), (1,)), ((), ())),
        preferred_element_type=jnp.float32,
    )

    @pl.when(k == 0)
    def _init():
        recon_ref[...] = partial + b_ref[...]

    @pl.when(k != 0)
    def _acc():
        recon_ref[...] += partial


def _decode(latents, thr, W_dec, b_dec, tb=512, kb=2048):
    grid = (N_TOK // tb, D_SAE // kb)
    return pl.pallas_call(
        _decode_body,
        grid=grid,
        in_specs=[
            pl.BlockSpec((tb, kb), lambda i, k: (i, k)),
            pl.BlockSpec((tb, 1), lambda i, k: (i, 0)),
            pl.BlockSpec((D_MODEL, kb), lambda i, k: (0, k)),
            pl.BlockSpec((1, D_MODEL), lambda i, k: (0, 0)),
        ],
        out_specs=[
            pl.BlockSpec((tb, kb), lambda i, k: (i, k)),
            pl.BlockSpec((tb, D_MODEL), lambda i, k: (i, 0)),
        ],
        out_shape=[
            jax.ShapeDtypeStruct((N_TOK, D_SAE), jnp.float32),
            jax.ShapeDtypeStruct((N_TOK, D_MODEL), jnp.float32),
        ],
        compiler_params=pltpu.CompilerParams(
            dimension_semantics=("arbitrary", "arbitrary"),
        ),
    )(latents, thr, W_dec.astype(jnp.bfloat16), b_dec.reshape(1, D_MODEL))


@jax.jit
def kernel(x, W_enc, b_enc, W_dec, b_dec):
    latents = _encode(x, W_enc, b_enc)
    thr = _sc_thresholds(latents)
    sparse_latents, recon = _decode(latents, thr, W_dec, b_dec)
    return recon, sparse_latents
